# baseline (device time: 93859 ns/iter reference)
import jax
import jax.numpy as jnp
from jax import lax
from jax.experimental import pallas as pl
from jax.experimental.pallas import tpu as pltpu

N_DEV = 16
S = 2


def kernel(x, w_mat):
    m, k_per = x.shape
    _, n = w_mat.shape
    m_per = m // N_DEV
    half = n // 2
    rs = m_per // S

    def body(x_ref, w_ref, out_ref, buf_r, buf_l, p_ref,
             send_r, send_l, recv_r, recv_l):
        my = lax.axis_index("i")

        def rho(rr):
            q = rr // 4
            t = rr % 4
            z = jnp.where(q % 2 == 0, t, 3 - t)
            return 4 * z + q

        def inv_rho(p):
            q = p % 4
            z = p // 4
            t = jnp.where(q % 2 == 0, z, 3 - z)
            return 4 * q + t

        r = inv_rho(my)
        right = rho(lax.rem(r + 1, N_DEV))
        left = rho(lax.rem(r - 1 + N_DEV, N_DEV))

        def chunk_r(h):
            return rho(lax.rem(r - 1 - h + 2 * N_DEV, N_DEV))

        def chunk_l(h):
            return rho(lax.rem(r + 1 + h, N_DEV))

        barrier_sem = pltpu.get_barrier_semaphore()
        for nbr in (left, right):
            pl.semaphore_signal(
                barrier_sem, inc=1,
                device_id=(nbr,), device_id_type=pl.DeviceIdType.MESH,
            )

        def stripe_gemm(c, sig, lo):
            xs = x_ref[pl.ds(c * m_per + sig * rs, rs), :]
            return jnp.dot(xs, w_ref[:, lo:lo + half],
                           preferred_element_type=jnp.float32)

        c_r0 = chunk_r(0)
        c_l0 = chunk_l(0)
        for sig in range(S):
            buf_r[0, pl.ds(sig * rs, rs), :] = stripe_gemm(c_r0, sig, 0)
            buf_l[0, pl.ds(sig * rs, rs), :] = stripe_gemm(c_l0, sig, half)

        pl.semaphore_wait(barrier_sem, 2)

        sends = []

        def send(h, sig, buf, ssems, rsems, tgt):
            d = pltpu.make_async_remote_copy(
                src_ref=buf.at[h, pl.ds(sig * rs, rs), :],
                dst_ref=buf.at[h + 1, pl.ds(sig * rs, rs), :],
                send_sem=ssems.at[h, sig],
                recv_sem=rsems.at[h + 1, sig],
                device_id=(tgt,),
                device_id_type=pl.DeviceIdType.MESH,
            )
            d.start()
            sends.append(d)

        def recv_wait(h, sig, buf, ssems, rsems):
            d = pltpu.make_async_remote_copy(
                src_ref=buf.at[h, pl.ds(sig * rs, rs), :],
                dst_ref=buf.at[h, pl.ds(sig * rs, rs), :],
                send_sem=ssems.at[0, sig],
                recv_sem=rsems.at[h, sig],
                device_id=(right,),
                device_id_type=pl.DeviceIdType.MESH,
            )
            d.wait_recv()

        for sig in range(S):
            send(0, sig, buf_r, send_r, recv_r, right)
            send(0, sig, buf_l, send_l, recv_l, left)

        p_ref[:, :] = jnp.dot(x_ref[:, :], w_ref[:, :],
                              preferred_element_type=jnp.float32)

        for h in range(1, N_DEV - 1):
            cr = chunk_r(h) * m_per
            cl = chunk_l(h) * m_per
            for sig in range(S):
                sl = pl.ds(sig * rs, rs)
                recv_wait(h, sig, buf_r, send_r, recv_r)
                send(h, sig, buf_r, send_r, recv_r, right)
                recv_wait(h, sig, buf_l, send_l, recv_l)
                send(h, sig, buf_l, send_l, recv_l, left)

        mine = my * m_per
        for sig in range(S):
            recv_wait(N_DEV - 1, sig, buf_r, send_r, recv_r)
            recv_wait(N_DEV - 1, sig, buf_l, send_l, recv_l)
        out_ref[:, 0:half] = jnp.maximum(
            buf_r[N_DEV - 1] + p_ref[pl.ds(mine, m_per), 0:half], 0.0)
        out_ref[:, half:n] = jnp.maximum(
            buf_l[N_DEV - 1] + p_ref[pl.ds(mine, m_per), half:n], 0.0)

        for d in sends:
            d.wait_send()

    return pl.pallas_call(
        body,
        out_shape=jax.ShapeDtypeStruct((m_per, n), jnp.float32),
        in_specs=[
            pl.BlockSpec(memory_space=pltpu.VMEM),
            pl.BlockSpec(memory_space=pltpu.VMEM),
        ],
        out_specs=pl.BlockSpec(memory_space=pltpu.VMEM),
        scratch_shapes=[
            pltpu.VMEM((N_DEV, m_per, half), jnp.float32),
            pltpu.VMEM((N_DEV, m_per, half), jnp.float32),
            pltpu.VMEM((m, n), jnp.float32),
            pltpu.SemaphoreType.DMA((N_DEV - 1, S)),
            pltpu.SemaphoreType.DMA((N_DEV - 1, S)),
            pltpu.SemaphoreType.DMA((N_DEV, S)),
            pltpu.SemaphoreType.DMA((N_DEV, S)),
        ],
        compiler_params=pltpu.CompilerParams(collective_id=0),
    )(x, w_mat)


# device time: 93787 ns/iter; 1.0008x vs baseline; 1.0008x over previous
import jax
import jax.numpy as jnp
from jax import lax
from jax.experimental import pallas as pl
from jax.experimental.pallas import tpu as pltpu

N_DEV = 16
S = 2


def kernel(x, w_mat):
    m, k_per = x.shape
    _, n = w_mat.shape
    m_per = m // N_DEV
    half = n // 2
    rs = m_per // S

    def body(x_ref, w_ref, out_ref, buf_r, buf_l, p_ref,
             send_r, send_l, recv_r, recv_l):
        my = lax.axis_index("i")

        def rho(rr):
            q = rr // 4
            t = rr % 4
            z = jnp.where(q % 2 == 0, t, 3 - t)
            return 4 * z + q

        def inv_rho(p):
            q = p % 4
            z = p // 4
            t = jnp.where(q % 2 == 0, z, 3 - z)
            return 4 * q + t

        r = inv_rho(my)
        right = rho(lax.rem(r + 1, N_DEV))
        left = rho(lax.rem(r - 1 + N_DEV, N_DEV))

        def chunk_r(h):
            return rho(lax.rem(r - 1 - h + 2 * N_DEV, N_DEV))

        def chunk_l(h):
            return rho(lax.rem(r + 1 + h, N_DEV))

        barrier_sem = pltpu.get_barrier_semaphore()
        for nbr in (left, right):
            pl.semaphore_signal(
                barrier_sem, inc=1,
                device_id=(nbr,), device_id_type=pl.DeviceIdType.MESH,
            )

        def stripe_gemm(c, sig, lo):
            xs = x_ref[pl.ds(c * m_per + sig * rs, rs), :]
            return jnp.dot(xs, w_ref[:, lo:lo + half],
                           preferred_element_type=jnp.float32)

        c_r0 = chunk_r(0)
        c_l0 = chunk_l(0)

        sends = []

        def send(h, sig, buf, ssems, rsems, tgt):
            d = pltpu.make_async_remote_copy(
                src_ref=buf.at[h, pl.ds(sig * rs, rs), :],
                dst_ref=buf.at[h + 1, pl.ds(sig * rs, rs), :],
                send_sem=ssems.at[h, sig],
                recv_sem=rsems.at[h + 1, sig],
                device_id=(tgt,),
                device_id_type=pl.DeviceIdType.MESH,
            )
            d.start()
            sends.append(d)

        def recv_wait(h, sig, buf, ssems, rsems):
            d = pltpu.make_async_remote_copy(
                src_ref=buf.at[h, pl.ds(sig * rs, rs), :],
                dst_ref=buf.at[h, pl.ds(sig * rs, rs), :],
                send_sem=ssems.at[0, sig],
                recv_sem=rsems.at[h, sig],
                device_id=(right,),
                device_id_type=pl.DeviceIdType.MESH,
            )
            d.wait_recv()

        buf_r[0, pl.ds(0, rs), :] = stripe_gemm(c_r0, 0, 0)
        buf_l[0, pl.ds(0, rs), :] = stripe_gemm(c_l0, 0, half)
        pl.semaphore_wait(barrier_sem, 2)
        send(0, 0, buf_r, send_r, recv_r, right)
        send(0, 0, buf_l, send_l, recv_l, left)
        for sig in range(1, S):
            buf_r[0, pl.ds(sig * rs, rs), :] = stripe_gemm(c_r0, sig, 0)
            send(0, sig, buf_r, send_r, recv_r, right)
            buf_l[0, pl.ds(sig * rs, rs), :] = stripe_gemm(c_l0, sig, half)
            send(0, sig, buf_l, send_l, recv_l, left)

        p_ref[:, :] = jnp.dot(x_ref[:, :], w_ref[:, :],
                              preferred_element_type=jnp.float32)

        for h in range(1, N_DEV - 1):
            cr = chunk_r(h) * m_per
            cl = chunk_l(h) * m_per
            for sig in range(S):
                sl = pl.ds(sig * rs, rs)
                recv_wait(h, sig, buf_r, send_r, recv_r)
                buf_r[h, sl, :] = (
                    buf_r[h, sl, :]
                    + p_ref[pl.ds(cr + sig * rs, rs), 0:half]
                )
                send(h, sig, buf_r, send_r, recv_r, right)
                recv_wait(h, sig, buf_l, send_l, recv_l)
                buf_l[h, sl, :] = (
                    buf_l[h, sl, :]
                    + p_ref[pl.ds(cl + sig * rs, rs), half:n]
                )
                send(h, sig, buf_l, send_l, recv_l, left)

        mine = my * m_per
        for sig in range(S):
            sl = pl.ds(sig * rs, rs)
            psl = pl.ds(mine + sig * rs, rs)
            recv_wait(N_DEV - 1, sig, buf_r, send_r, recv_r)
            out_ref[sl, 0:half] = jnp.maximum(
                buf_r[N_DEV - 1, sl, :] + p_ref[psl, 0:half], 0.0)
            recv_wait(N_DEV - 1, sig, buf_l, send_l, recv_l)
            out_ref[sl, half:n] = jnp.maximum(
                buf_l[N_DEV - 1, sl, :] + p_ref[psl, half:n], 0.0)

        for d in sends:
            d.wait_send()

    return pl.pallas_call(
        body,
        out_shape=jax.ShapeDtypeStruct((m_per, n), jnp.float32),
        in_specs=[
            pl.BlockSpec(memory_space=pltpu.VMEM),
            pl.BlockSpec(memory_space=pltpu.VMEM),
        ],
        out_specs=pl.BlockSpec(memory_space=pltpu.VMEM),
        scratch_shapes=[
            pltpu.VMEM((N_DEV, m_per, half), jnp.float32),
            pltpu.VMEM((N_DEV, m_per, half), jnp.float32),
            pltpu.VMEM((m, n), jnp.float32),
            pltpu.SemaphoreType.DMA((N_DEV - 1, S)),
            pltpu.SemaphoreType.DMA((N_DEV - 1, S)),
            pltpu.SemaphoreType.DMA((N_DEV, S)),
            pltpu.SemaphoreType.DMA((N_DEV, S)),
        ],
        compiler_params=pltpu.CompilerParams(collective_id=0),
    )(x, w_mat)


# device time: 57022 ns/iter; 1.6460x vs baseline; 1.6448x over previous
import jax
import jax.numpy as jnp
from jax import lax
from jax.experimental import pallas as pl
from jax.experimental.pallas import tpu as pltpu

N_DEV = 16
S = 2


def kernel(x, w_mat):
    m, k_per = x.shape
    _, n = w_mat.shape
    m_per = m // N_DEV
    half = n // 2
    rs = m_per // S

    def body(x_ref, w_ref, out_ref, buf_r, buf_l, p_ref,
             send_r, send_l, recv_r, recv_l):
        my = lax.axis_index("i")

        def rho(rr):
            q = rr // 4
            t = rr % 4
            z = jnp.where(q % 2 == 0, t, 3 - t)
            return 4 * z + q

        def inv_rho(p):
            q = p % 4
            z = p // 4
            t = jnp.where(q % 2 == 0, z, 3 - z)
            return 4 * q + t

        r = inv_rho(my)
        right = rho(lax.rem(r + 1, N_DEV))
        left = rho(lax.rem(r - 1 + N_DEV, N_DEV))

        def chunk_r(h):
            return rho(lax.rem(r - 1 - h + 2 * N_DEV, N_DEV))

        def chunk_l(h):
            return rho(lax.rem(r + 1 + h, N_DEV))

        barrier_sem = pltpu.get_barrier_semaphore()
        for nbr in (left, right):
            pl.semaphore_signal(
                barrier_sem, inc=1,
                device_id=(nbr,), device_id_type=pl.DeviceIdType.MESH,
            )

        def stripe_gemm(c, sig, lo):
            xs = x_ref[pl.ds(c * m_per + sig * rs, rs), :]
            return jnp.dot(xs, w_ref[:, lo:lo + half],
                           preferred_element_type=jnp.float32)

        c_r0 = chunk_r(0)
        c_l0 = chunk_l(0)

        sends = []

        def send(h, sig, buf, ssems, rsems, tgt):
            d = pltpu.make_async_remote_copy(
                src_ref=buf.at[h, pl.ds(sig * rs, rs), :],
                dst_ref=buf.at[h + 1, pl.ds(sig * rs, rs), :],
                send_sem=ssems.at[h, sig],
                recv_sem=rsems.at[h + 1, sig],
                device_id=(tgt,),
                device_id_type=pl.DeviceIdType.MESH,
            )
            d.start()
            sends.append(d)

        def recv_wait(h, sig, buf, ssems, rsems):
            d = pltpu.make_async_remote_copy(
                src_ref=buf.at[h, pl.ds(sig * rs, rs), :],
                dst_ref=buf.at[h, pl.ds(sig * rs, rs), :],
                send_sem=ssems.at[0, sig],
                recv_sem=rsems.at[h, sig],
                device_id=(right,),
                device_id_type=pl.DeviceIdType.MESH,
            )
            d.wait_recv()

        buf_r[0, pl.ds(0, rs), :] = stripe_gemm(c_r0, 0, 0).astype(jnp.bfloat16)
        buf_l[0, pl.ds(0, rs), :] = stripe_gemm(c_l0, 0, half).astype(jnp.bfloat16)
        pl.semaphore_wait(barrier_sem, 2)
        send(0, 0, buf_r, send_r, recv_r, right)
        send(0, 0, buf_l, send_l, recv_l, left)
        for sig in range(1, S):
            buf_r[0, pl.ds(sig * rs, rs), :] = stripe_gemm(c_r0, sig, 0).astype(jnp.bfloat16)
            send(0, sig, buf_r, send_r, recv_r, right)
            buf_l[0, pl.ds(sig * rs, rs), :] = stripe_gemm(c_l0, sig, half).astype(jnp.bfloat16)
            send(0, sig, buf_l, send_l, recv_l, left)

        p_ref[:, :] = jnp.dot(x_ref[:, :], w_ref[:, :],
                              preferred_element_type=jnp.float32)

        for h in range(1, N_DEV - 1):
            cr = chunk_r(h) * m_per
            cl = chunk_l(h) * m_per
            for sig in range(S):
                sl = pl.ds(sig * rs, rs)
                recv_wait(h, sig, buf_r, send_r, recv_r)
                buf_r[h, sl, :] = (
                    buf_r[h, sl, :].astype(jnp.float32)
                    + p_ref[pl.ds(cr + sig * rs, rs), 0:half]
                ).astype(jnp.bfloat16)
                send(h, sig, buf_r, send_r, recv_r, right)
                recv_wait(h, sig, buf_l, send_l, recv_l)
                buf_l[h, sl, :] = (
                    buf_l[h, sl, :].astype(jnp.float32)
                    + p_ref[pl.ds(cl + sig * rs, rs), half:n]
                ).astype(jnp.bfloat16)
                send(h, sig, buf_l, send_l, recv_l, left)

        mine = my * m_per
        for sig in range(S):
            sl = pl.ds(sig * rs, rs)
            psl = pl.ds(mine + sig * rs, rs)
            recv_wait(N_DEV - 1, sig, buf_r, send_r, recv_r)
            out_ref[sl, 0:half] = jnp.maximum(
                buf_r[N_DEV - 1, sl, :].astype(jnp.float32) + p_ref[psl, 0:half], 0.0)
            recv_wait(N_DEV - 1, sig, buf_l, send_l, recv_l)
            out_ref[sl, half:n] = jnp.maximum(
                buf_l[N_DEV - 1, sl, :].astype(jnp.float32) + p_ref[psl, half:n], 0.0)

        for d in sends:
            d.wait_send()

    return pl.pallas_call(
        body,
        out_shape=jax.ShapeDtypeStruct((m_per, n), jnp.float32),
        in_specs=[
            pl.BlockSpec(memory_space=pltpu.VMEM),
            pl.BlockSpec(memory_space=pltpu.VMEM),
        ],
        out_specs=pl.BlockSpec(memory_space=pltpu.VMEM),
        scratch_shapes=[
            pltpu.VMEM((N_DEV, m_per, half), jnp.bfloat16),
            pltpu.VMEM((N_DEV, m_per, half), jnp.bfloat16),
            pltpu.VMEM((m, n), jnp.float32),
            pltpu.SemaphoreType.DMA((N_DEV - 1, S)),
            pltpu.SemaphoreType.DMA((N_DEV - 1, S)),
            pltpu.SemaphoreType.DMA((N_DEV, S)),
            pltpu.SemaphoreType.DMA((N_DEV, S)),
        ],
        compiler_params=pltpu.CompilerParams(collective_id=0),
    )(x, w_mat)


# device time: 53876 ns/iter; 1.7421x vs baseline; 1.0584x over previous
import jax
import jax.numpy as jnp
from jax import lax
from jax.experimental import pallas as pl
from jax.experimental.pallas import tpu as pltpu

N_DEV = 16
S = 4


def kernel(x, w_mat):
    m, k_per = x.shape
    _, n = w_mat.shape
    m_per = m // N_DEV
    half = n // 2
    rs = m_per // S

    def body(x_ref, w_ref, out_ref, buf_r, buf_l, p_ref,
             send_r, send_l, recv_r, recv_l):
        my = lax.axis_index("i")

        def rho(rr):
            q = rr // 4
            t = rr % 4
            z = jnp.where(q % 2 == 0, t, 3 - t)
            return 4 * z + q

        def inv_rho(p):
            q = p % 4
            z = p // 4
            t = jnp.where(q % 2 == 0, z, 3 - z)
            return 4 * q + t

        r = inv_rho(my)
        right = rho(lax.rem(r + 1, N_DEV))
        left = rho(lax.rem(r - 1 + N_DEV, N_DEV))

        def chunk_r(h):
            return rho(lax.rem(r - 1 - h + 2 * N_DEV, N_DEV))

        def chunk_l(h):
            return rho(lax.rem(r + 1 + h, N_DEV))

        barrier_sem = pltpu.get_barrier_semaphore()
        for nbr in (left, right):
            pl.semaphore_signal(
                barrier_sem, inc=1,
                device_id=(nbr,), device_id_type=pl.DeviceIdType.MESH,
            )

        def stripe_gemm(c, sig, lo):
            xs = x_ref[pl.ds(c * m_per + sig * rs, rs), :]
            return jnp.dot(xs, w_ref[:, lo:lo + half],
                           preferred_element_type=jnp.float32)

        c_r0 = chunk_r(0)
        c_l0 = chunk_l(0)

        sends = []

        def send(h, sig, buf, ssems, rsems, tgt):
            d = pltpu.make_async_remote_copy(
                src_ref=buf.at[h, pl.ds(sig * rs, rs), :],
                dst_ref=buf.at[h + 1, pl.ds(sig * rs, rs), :],
                send_sem=ssems.at[h, sig],
                recv_sem=rsems.at[h + 1, sig],
                device_id=(tgt,),
                device_id_type=pl.DeviceIdType.MESH,
            )
            d.start()
            sends.append(d)

        def recv_wait(h, sig, buf, ssems, rsems):
            d = pltpu.make_async_remote_copy(
                src_ref=buf.at[h, pl.ds(sig * rs, rs), :],
                dst_ref=buf.at[h, pl.ds(sig * rs, rs), :],
                send_sem=ssems.at[0, sig],
                recv_sem=rsems.at[h, sig],
                device_id=(right,),
                device_id_type=pl.DeviceIdType.MESH,
            )
            d.wait_recv()

        buf_r[0, pl.ds(0, rs), :] = stripe_gemm(c_r0, 0, 0).astype(jnp.bfloat16)
        buf_l[0, pl.ds(0, rs), :] = stripe_gemm(c_l0, 0, half).astype(jnp.bfloat16)
        pl.semaphore_wait(barrier_sem, 2)
        send(0, 0, buf_r, send_r, recv_r, right)
        send(0, 0, buf_l, send_l, recv_l, left)
        for sig in range(1, S):
            buf_r[0, pl.ds(sig * rs, rs), :] = stripe_gemm(c_r0, sig, 0).astype(jnp.bfloat16)
            send(0, sig, buf_r, send_r, recv_r, right)
            buf_l[0, pl.ds(sig * rs, rs), :] = stripe_gemm(c_l0, sig, half).astype(jnp.bfloat16)
            send(0, sig, buf_l, send_l, recv_l, left)

        p_ref[:, :] = jnp.dot(x_ref[:, :], w_ref[:, :],
                              preferred_element_type=jnp.float32)

        for h in range(1, N_DEV - 1):
            cr = chunk_r(h) * m_per
            cl = chunk_l(h) * m_per
            for sig in range(S):
                sl = pl.ds(sig * rs, rs)
                recv_wait(h, sig, buf_r, send_r, recv_r)
                buf_r[h, sl, :] = (
                    buf_r[h, sl, :].astype(jnp.float32)
                    + p_ref[pl.ds(cr + sig * rs, rs), 0:half]
                ).astype(jnp.bfloat16)
                send(h, sig, buf_r, send_r, recv_r, right)
                recv_wait(h, sig, buf_l, send_l, recv_l)
                buf_l[h, sl, :] = (
                    buf_l[h, sl, :].astype(jnp.float32)
                    + p_ref[pl.ds(cl + sig * rs, rs), half:n]
                ).astype(jnp.bfloat16)
                send(h, sig, buf_l, send_l, recv_l, left)

        mine = my * m_per
        for sig in range(S):
            sl = pl.ds(sig * rs, rs)
            psl = pl.ds(mine + sig * rs, rs)
            recv_wait(N_DEV - 1, sig, buf_r, send_r, recv_r)
            out_ref[sl, 0:half] = jnp.maximum(
                buf_r[N_DEV - 1, sl, :].astype(jnp.float32) + p_ref[psl, 0:half], 0.0)
            recv_wait(N_DEV - 1, sig, buf_l, send_l, recv_l)
            out_ref[sl, half:n] = jnp.maximum(
                buf_l[N_DEV - 1, sl, :].astype(jnp.float32) + p_ref[psl, half:n], 0.0)

        for d in sends:
            d.wait_send()

    return pl.pallas_call(
        body,
        out_shape=jax.ShapeDtypeStruct((m_per, n), jnp.float32),
        in_specs=[
            pl.BlockSpec(memory_space=pltpu.VMEM),
            pl.BlockSpec(memory_space=pltpu.VMEM),
        ],
        out_specs=pl.BlockSpec(memory_space=pltpu.VMEM),
        scratch_shapes=[
            pltpu.VMEM((N_DEV, m_per, half), jnp.bfloat16),
            pltpu.VMEM((N_DEV, m_per, half), jnp.bfloat16),
            pltpu.VMEM((m, n), jnp.float32),
            pltpu.SemaphoreType.DMA((N_DEV - 1, S)),
            pltpu.SemaphoreType.DMA((N_DEV - 1, S)),
            pltpu.SemaphoreType.DMA((N_DEV, S)),
            pltpu.SemaphoreType.DMA((N_DEV, S)),
        ],
        compiler_params=pltpu.CompilerParams(collective_id=0),
    )(x, w_mat)
